# baseline (speedup 1.0000x reference)
import functools

import jax
import jax.numpy as jnp
from jax import lax
from jax.experimental import pallas as pl
from jax.experimental.pallas import tpu as pltpu
from jax.experimental.pallas import tpu_sc as plsc

B = 16384
DD = 13
DS = 26
NF = DD + DS
H = 32
L = 16

NW = 32
BPW = B // NW
CBL = 128
CHUNKS = BPW // CBL
NBLK = CBL // L
TV = 1000000
TP = TV // 4


RSLAB = 64
NSLAB_FULL = TP // RSLAB
RS_BASE = 122
RS_LEFT = TP - NSLAB_FULL * RSLAB


def _repack_body(tbl_hbm, out_hbm, in_v, out_v, sem_i, sem_ou):
    wid = lax.axis_index("s") * 2 + lax.axis_index("c")
    s0 = wid * RS_BASE + jnp.minimum(wid, 2)
    nslab = RS_BASE + (wid < 2).astype(jnp.int32)

    def fire_in(i, b):
        pltpu.async_copy(
            tbl_hbm.at[pl.ds(pl.multiple_of((s0 + i) * (RSLAB * 4), 8),
                             RSLAB * 4), :],
            in_v.at[b], sem_i)

    def drain_in(i, b):
        pltpu.make_async_copy(
            tbl_hbm.at[pl.ds(pl.multiple_of((s0 + i) * (RSLAB * 4), 8),
                             RSLAB * 4), :],
            in_v.at[b], sem_i).wait()

    def wait_out():
        pltpu.make_async_copy(
            out_v.at[0], out_hbm.at[pl.ds(0, RSLAB), :], sem_ou).wait()

    fire_in(0, 0)
    fire_in(1, 1)

    @pl.loop(0, RS_BASE)
    def _slab(i):
        b = i % 2
        drain_in(i, b)

        @pl.when(i >= 2)
        def _():
            wait_out()

        @pl.loop(0, RSLAB)
        def _row(p):
            for k in range(4):
                for hh in range(0, H, L):
                    out_v[b, p, pl.ds(32 * k + hh, L)] = (
                        in_v[b, 4 * p + k, pl.ds(hh, L)])

        @pl.when(i + 2 < nslab)
        def _():
            fire_in(i + 2, b)

        pltpu.async_copy(
            out_v.at[b],
            out_hbm.at[pl.ds(pl.multiple_of((s0 + i) * RSLAB, 8), RSLAB), :],
            sem_ou)

    @pl.when(nslab > RS_BASE)
    def _():
        i = RS_BASE
        b = i % 2
        drain_in(i, b)
        wait_out()

        @pl.loop(0, RSLAB)
        def _row(p):
            for k in range(4):
                for hh in range(0, H, L):
                    out_v[b, p, pl.ds(32 * k + hh, L)] = (
                        in_v[b, 4 * p + k, pl.ds(hh, L)])

        pltpu.async_copy(
            out_v.at[b],
            out_hbm.at[pl.ds(pl.multiple_of((s0 + i) * RSLAB, 8), RSLAB), :],
            sem_ou)

    @pl.when(wid == 2)
    def _():
        wait_out()
        wait_out()
        pltpu.sync_copy(
            tbl_hbm.at[pl.ds(NSLAB_FULL * RSLAB * 4, RS_LEFT * 4), :],
            in_v.at[0, pl.ds(0, RS_LEFT * 4), :])

        @pl.loop(0, RS_LEFT)
        def _row(p):
            for k in range(4):
                for hh in range(0, H, L):
                    out_v[0, p, pl.ds(32 * k + hh, L)] = (
                        in_v[0, 4 * p + k, pl.ds(hh, L)])

        pltpu.sync_copy(
            out_v.at[0, pl.ds(0, RS_LEFT), :],
            out_hbm.at[pl.ds(NSLAB_FULL * RSLAB, RS_LEFT), :])

    @pl.when(wid != 2)
    def _():
        wait_out()
        wait_out()


@functools.cache
def _sc_repack():
    mesh = plsc.VectorSubcoreMesh(core_axis_name="c", subcore_axis_name="s")
    return functools.partial(
        pl.kernel,
        out_type=jax.ShapeDtypeStruct((TP, 128), jnp.float32),
        mesh=mesh,
        scratch_types=[
            pltpu.VMEM((2, RSLAB * 4, H), jnp.float32),
            pltpu.VMEM((2, RSLAB, 128), jnp.float32),
            pltpu.SemaphoreType.DMA,
            pltpu.SemaphoreType.DMA,
        ],
        compiler_params=pltpu.CompilerParams(
            use_tc_tiling_on_sc=True, needs_layout_passes=False),
    )(_repack_body)


def _body(x_hbm, idx_hbm, w_hbm, tbl_hbm, col_hbm, out_hbm,
          idx_v, qidx_v, x_v, gath_v, plane_v, df_v, w_v, col_v,
          sem_g, sem_od, sem_os):
    wid = lax.axis_index("s") * 2 + lax.axis_index("c")
    base = wid * BPW

    pltpu.sync_copy(w_hbm, w_v)
    pltpu.sync_copy(col_hbm, col_v)
    iota = jax.lax.iota(jnp.int32, L)
    iota128 = iota * 128

    def fire_gather(s):
        pltpu.async_copy(tbl_hbm.at[qidx_v.at[s]], gath_v.at[s % 3], sem_g)

    def drain_gather(s):
        pltpu.make_async_copy(
            tbl_hbm.at[qidx_v.at[s]], gath_v.at[s % 3], sem_g).wait()

    @pl.loop(0, CHUNKS)
    def _chunk(ci):
        c0 = pl.multiple_of(base + ci * CBL, 128)

        pltpu.sync_copy(idx_hbm.at[:, pl.ds(c0, CBL)], idx_v)
        pltpu.sync_copy(x_hbm.at[:, pl.ds(c0, CBL)], x_v)

        @pl.loop(0, DS)
        def _q(s):
            for blk in range(NBLK):
                v = idx_v[s, pl.ds(blk * L, L)]
                qidx_v[s, pl.ds(blk * L, L)] = lax.shift_right_logical(v, 2)

        fire_gather(0)
        fire_gather(1)
        fire_gather(2)

        @pl.loop(0, DD)
        def _dense(f):
            fp = f % 2

            @pl.when(f >= 2)
            def _():
                pltpu.make_async_copy(
                    df_v.at[0], out_hbm.at[0, :, pl.ds(0, CBL)],
                    sem_od).wait()

            for hh in range(0, H, L):
                wrow = w_v[f, pl.ds(hh, L)]
                crow = col_v[f, pl.ds(hh, L)]
                for j in range(L):
                    w = wrow[j]
                    c = crow[j]
                    for blk in range(NBLK):
                        xb = x_v[f, pl.ds(blk * L, L)]
                        df_v[fp, hh + j, pl.ds(blk * L, L)] = (
                            jnp.maximum(xb * w, 0.0) + c)

            pltpu.async_copy(
                df_v.at[fp], out_hbm.at[f, :, pl.ds(c0, CBL)], sem_od)

        @pl.loop(0, DS)
        def _sparse(s):
            par = s % 3
            pp = s % 2
            drain_gather(s)

            @pl.when(s >= 2)
            def _():
                pltpu.make_async_copy(
                    plane_v.at[0], out_hbm.at[DD, :, pl.ds(0, CBL)],
                    sem_os).wait()

            gv = gath_v.at[par]
            for blk in range(NBLK):
                idxb = idx_v[s, pl.ds(blk * L, L)]
                colv0 = (idxb & 3) << 5
                rows = iota + blk * L
                for hh in range(0, H, L):
                    crow = col_v[DD + s, pl.ds(hh, L)]
                    for j in range(L):
                        g = plsc.load_gather(gv, [rows, colv0 + (hh + j)])
                        plane_v[pp, hh + j, pl.ds(blk * L, L)] = g + crow[j]

            @pl.when(s + 3 < DS)
            def _():
                fire_gather(s + 3)

            pltpu.async_copy(
                plane_v.at[pp], out_hbm.at[DD + s, :, pl.ds(c0, CBL)],
                sem_os)

        for _ in range(2):
            pltpu.make_async_copy(
                df_v.at[0], out_hbm.at[0, :, pl.ds(0, CBL)], sem_od).wait()
            pltpu.make_async_copy(
                plane_v.at[0], out_hbm.at[DD, :, pl.ds(0, CBL)],
                sem_os).wait()


@functools.cache
def _sc_embed():
    mesh = plsc.VectorSubcoreMesh(core_axis_name="c", subcore_axis_name="s")
    return functools.partial(
        pl.kernel,
        out_type=jax.ShapeDtypeStruct((NF, H, B), jnp.float32),
        mesh=mesh,
        scratch_types=[
            pltpu.VMEM((DS, CBL), jnp.int32),
            pltpu.VMEM((DS, CBL), jnp.int32),
            pltpu.VMEM((DD, CBL), jnp.float32),
            pltpu.VMEM((3, CBL, 128), jnp.float32),
            pltpu.VMEM((2, H, CBL), jnp.float32),
            pltpu.VMEM((2, H, CBL), jnp.float32),
            pltpu.VMEM((DD, H), jnp.float32),
            pltpu.VMEM((NF, H), jnp.float32),
            pltpu.SemaphoreType.DMA,
            pltpu.SemaphoreType.DMA,
            pltpu.SemaphoreType.DMA,
        ],
        compiler_params=pltpu.CompilerParams(
            use_tc_tiling_on_sc=True, needs_layout_passes=False),
    )(_body)


def kernel(x_dense, x_sparse, dense_embed_weight, sparse_embed_weight,
           col_embed):
    xt = x_dense.T
    idxt = x_sparse.astype(jnp.int32).T
    tbl = _sc_repack()(sparse_embed_weight)
    out = _sc_embed()(xt, idxt, dense_embed_weight, tbl, col_embed)
    return jnp.transpose(out, (2, 0, 1))


# speedup vs baseline: 1.0732x; 1.0732x over previous
import functools

import jax
import jax.numpy as jnp
from jax import lax
from jax.experimental import pallas as pl
from jax.experimental.pallas import tpu as pltpu
from jax.experimental.pallas import tpu_sc as plsc

B = 16384
DD = 13
DS = 26
NF = DD + DS
H = 32
L = 16

NW = 32
BPW = B // NW
CBL = 128
CHUNKS = BPW // CBL
NBLK = CBL // L
TV = 1000000
TP = TV // 4


def _body(x_hbm, idx_hbm, w_hbm, tbl_hbm, col_hbm, out_hbm,
          idx_v, x_v, gath_v, plane_v, df_v, w_v, col_v,
          sem_g, sem_od, sem_os):
    wid = lax.axis_index("s") * 2 + lax.axis_index("c")
    base = wid * BPW

    pltpu.sync_copy(w_hbm, w_v)
    pltpu.sync_copy(col_hbm, col_v)
    iota = jax.lax.iota(jnp.int32, L)
    iota128 = iota * 128

    def fire_gather(s):
        pltpu.async_copy(tbl_hbm.at[idx_v.at[s]], gath_v.at[s % 3], sem_g)

    def drain_gather(s):
        pltpu.make_async_copy(
            tbl_hbm.at[idx_v.at[s]], gath_v.at[s % 3], sem_g).wait()

    @pl.loop(0, CHUNKS)
    def _chunk(ci):
        c0 = pl.multiple_of(base + ci * CBL, 128)

        pltpu.sync_copy(idx_hbm.at[:, pl.ds(c0, CBL)], idx_v)
        pltpu.sync_copy(x_hbm.at[:, pl.ds(c0, CBL)], x_v)

        fire_gather(0)
        fire_gather(1)
        fire_gather(2)

        @pl.loop(0, DD)
        def _dense(f):
            fp = f % 2

            @pl.when(f >= 2)
            def _():
                pltpu.make_async_copy(
                    df_v.at[0], out_hbm.at[0, :, pl.ds(0, CBL)],
                    sem_od).wait()

            for hh in range(0, H, L):
                wrow = w_v[f, pl.ds(hh, L)]
                crow = col_v[f, pl.ds(hh, L)]
                for j in range(L):
                    w = wrow[j]
                    c = crow[j]
                    for blk in range(NBLK):
                        xb = x_v[f, pl.ds(blk * L, L)]
                        df_v[fp, hh + j, pl.ds(blk * L, L)] = (
                            jnp.maximum(xb * w, 0.0) + c)

            pltpu.async_copy(
                df_v.at[fp], out_hbm.at[f, :, pl.ds(c0, CBL)], sem_od)

        @pl.loop(0, DS)
        def _sparse(s):
            par = s % 3
            pp = s % 2
            drain_gather(s)

            @pl.when(s >= 2)
            def _():
                pltpu.make_async_copy(
                    plane_v.at[0], out_hbm.at[DD, :, pl.ds(0, CBL)],
                    sem_os).wait()

            gv = gath_v.at[par]
            for blk in range(NBLK):
                rows = iota + blk * L
                for hh in range(0, H, L):
                    crow = col_v[DD + s, pl.ds(hh, L)]
                    for j in range(L):
                        hvec = jnp.full((L,), hh + j, dtype=jnp.int32)
                        g = plsc.load_gather(gv, [rows, hvec])
                        plane_v[pp, hh + j, pl.ds(blk * L, L)] = g + crow[j]

            @pl.when(s + 3 < DS)
            def _():
                fire_gather(s + 3)

            pltpu.async_copy(
                plane_v.at[pp], out_hbm.at[DD + s, :, pl.ds(c0, CBL)],
                sem_os)

        for _ in range(2):
            pltpu.make_async_copy(
                df_v.at[0], out_hbm.at[0, :, pl.ds(0, CBL)], sem_od).wait()
            pltpu.make_async_copy(
                plane_v.at[0], out_hbm.at[DD, :, pl.ds(0, CBL)],
                sem_os).wait()


@functools.cache
def _sc_embed():
    mesh = plsc.VectorSubcoreMesh(core_axis_name="c", subcore_axis_name="s")
    return functools.partial(
        pl.kernel,
        out_type=jax.ShapeDtypeStruct((NF, H, B), jnp.float32),
        mesh=mesh,
        scratch_types=[
            pltpu.VMEM((DS, CBL), jnp.int32),
            pltpu.VMEM((DD, CBL), jnp.float32),
            pltpu.VMEM((3, CBL, 128), jnp.float32),
            pltpu.VMEM((2, H, CBL), jnp.float32),
            pltpu.VMEM((2, H, CBL), jnp.float32),
            pltpu.VMEM((DD, H), jnp.float32),
            pltpu.VMEM((NF, H), jnp.float32),
            pltpu.SemaphoreType.DMA,
            pltpu.SemaphoreType.DMA,
            pltpu.SemaphoreType.DMA,
        ],
        compiler_params=pltpu.CompilerParams(
            use_tc_tiling_on_sc=True, needs_layout_passes=False),
    )(_body)


def kernel(x_dense, x_sparse, dense_embed_weight, sparse_embed_weight,
           col_embed):
    xt = x_dense.T
    idxt = x_sparse.astype(jnp.int32).T
    tblp = jnp.pad(sparse_embed_weight, ((0, 0), (0, 128 - H)))
    out = _sc_embed()(xt, idxt, dense_embed_weight, tblp, col_embed)
    return jnp.transpose(out, (2, 0, 1))
